# baseline (speedup 1.0000x reference)
import jax
import jax.numpy as jnp
from jax import lax
from jax.experimental import pallas as pl
from jax.experimental.pallas import tpu as pltpu
from jax.experimental.pallas import tpu_sc as plsc

B, S, H, D, K = 64, 32, 256, 64, 8
L = 16
NC, NS = 2, 16
NW = NC * NS
P = B * K
PPW = P // NW
RPB = 8
SUB = (H * D) // RPB
VROWS = B * S * RPB
PPD = 2
NDMA = PPW // PPD
NSLOT = 3


def _gather_body(mem_hbm, tk_hbm, out_hbm, tk_v, idx_v, bufs, gsems, osems):
    wid = lax.axis_index("s") * NC + lax.axis_index("c")
    base = wid * PPW

    pltpu.sync_copy(tk_hbm.at[pl.ds(base, PPW)], tk_v)
    lane = lax.broadcasted_iota(jnp.int32, (L,), 0)
    p_vec = base + lane
    b_vec = lax.shift_right_logical(p_vec, 3)
    flat_reg = b_vec * S + tk_v[...]
    alt = lane & 1
    for t in range(PPW // 2):
        fvec = flat_reg.at[2 * t + alt].get(mode="promise_in_bounds")
        idx_v[pl.ds(t * L, L)] = fvec

    def gissue(g):
        slot = g % NSLOT
        return pltpu.async_copy(
            mem_hbm.at[idx_v.at[pl.ds(g * L, PPD)]], bufs[slot], gsems[slot])

    gcp = [None] * NSLOT
    ocp = [None] * NSLOT
    for g in range(NSLOT):
        gcp[g] = gissue(g)
    for g in range(NDMA):
        slot = g % NSLOT
        gcp[slot].wait()
        ocp[slot] = pltpu.async_copy(
            bufs[slot], out_hbm.at[pl.ds(base + PPD * g, PPD)], osems[slot])
        if g + NSLOT < NDMA:
            ocp[slot].wait()
            gcp[slot] = gissue(g + NSLOT)
    for g in range(NDMA - NSLOT, NDMA):
        ocp[g % NSLOT].wait()


def kernel(memories_per_batch, topk_indices, gates):
    del gates
    mem6 = memories_per_batch.reshape(B, S, 2, 128, 8, 8)
    mem2 = mem6.transpose(0, 1, 5, 4, 2, 3).reshape(B * S, RPB, SUB)
    tk = topk_indices.reshape(P).astype(jnp.int32)

    mesh = plsc.VectorSubcoreMesh(
        core_axis_name="c", subcore_axis_name="s", num_cores=NC, num_subcores=NS
    )
    out = pl.kernel(
        _gather_body,
        out_type=jax.ShapeDtypeStruct((P, RPB, SUB), jnp.float32),
        mesh=mesh,
        scratch_types=[
            pltpu.VMEM((PPW,), jnp.int32),
            pltpu.VMEM((PPW * RPB,), jnp.int32),
            [pltpu.VMEM((PPD, RPB, SUB), jnp.float32) for _ in range(NSLOT)],
            [pltpu.SemaphoreType.DMA for _ in range(NSLOT)],
            [pltpu.SemaphoreType.DMA for _ in range(NSLOT)],
        ],
    )(mem2, tk)
    out6 = out.reshape(B, K, 8, 8, 2, 128)
    return out6.transpose(0, 1, 4, 5, 3, 2).reshape(B, K, H, D)


# speedup vs baseline: 1.0034x; 1.0034x over previous
import jax
import jax.numpy as jnp
from jax import lax
from jax.experimental import pallas as pl
from jax.experimental.pallas import tpu as pltpu
from jax.experimental.pallas import tpu_sc as plsc

B, S, H, D, K = 64, 32, 256, 64, 8
L = 16
NC, NS = 2, 16
NW = NC * NS
P = B * K
PPW = P // NW
RPB = 8
SUB = (H * D) // RPB
VROWS = B * S * RPB
PPD = 1
NDMA = PPW // PPD
NSLOT = 5


def _gather_body(mem_hbm, tk_hbm, out_hbm, tk_v, idx_v, bufs, gsems, osems):
    wid = lax.axis_index("s") * NC + lax.axis_index("c")
    base = wid * PPW

    pltpu.sync_copy(tk_hbm.at[pl.ds(base, PPW)], tk_v)
    lane = lax.broadcasted_iota(jnp.int32, (L,), 0)
    p_vec = base + lane
    b_vec = lax.shift_right_logical(p_vec, 3)
    flat_reg = b_vec * S + tk_v[...]
    half = lax.shift_right_logical(lane, 3)
    for t in range(PPW // 2):
        fvec = flat_reg.at[2 * t + half].get(mode="promise_in_bounds")
        idx_v[pl.ds(t * L, L)] = fvec

    def gissue(g):
        slot = g % NSLOT
        return pltpu.async_copy(
            mem_hbm.at[idx_v.at[pl.ds(g * RPB, 1)]], bufs[slot], gsems[slot])

    gcp = [None] * NSLOT
    ocp = [None] * NSLOT
    for g in range(NSLOT):
        gcp[g] = gissue(g)
    for g in range(NDMA):
        slot = g % NSLOT
        gcp[slot].wait()
        ocp[slot] = pltpu.async_copy(
            bufs[slot], out_hbm.at[pl.ds(base + g, 1)], osems[slot])
        if g + NSLOT < NDMA:
            ocp[slot].wait()
            gcp[slot] = gissue(g + NSLOT)
    for g in range(NDMA - NSLOT, NDMA):
        ocp[g % NSLOT].wait()


def kernel(memories_per_batch, topk_indices, gates):
    del gates
    mem6 = memories_per_batch.reshape(B, S, 2, 128, 8, 8)
    mem2 = mem6.transpose(0, 1, 5, 4, 2, 3).reshape(B * S, RPB, SUB)
    tk = topk_indices.reshape(P).astype(jnp.int32)

    mesh = plsc.VectorSubcoreMesh(
        core_axis_name="c", subcore_axis_name="s", num_cores=NC, num_subcores=NS
    )
    out = pl.kernel(
        _gather_body,
        out_type=jax.ShapeDtypeStruct((P, RPB, SUB), jnp.float32),
        mesh=mesh,
        scratch_types=[
            pltpu.VMEM((PPW,), jnp.int32),
            pltpu.VMEM((PPW * RPB,), jnp.int32),
            [pltpu.VMEM((1, RPB, SUB), jnp.float32) for _ in range(NSLOT)],
            [pltpu.SemaphoreType.DMA for _ in range(NSLOT)],
            [pltpu.SemaphoreType.DMA for _ in range(NSLOT)],
        ],
    )(mem2, tk)
    out6 = out.reshape(B, K, 8, 8, 2, 128)
    return out6.transpose(0, 1, 4, 5, 3, 2).reshape(B, K, H, D)


# speedup vs baseline: 1.0065x; 1.0030x over previous
import jax
import jax.numpy as jnp
from jax import lax
from jax.experimental import pallas as pl
from jax.experimental.pallas import tpu as pltpu
from jax.experimental.pallas import tpu_sc as plsc

B, S, H, D, K = 64, 32, 256, 64, 8
L = 16
NC, NS = 2, 16
NW = NC * NS
P = B * K
PPW = P // NW
RPB = 8
SUB = (H * D) // RPB
PPD = 1
NDMA = PPW // PPD
NSLOT = 5


def _gather_body(mem_hbm, tk_hbm, out_hbm, tk_v, idx_v, bufs, gsems, osems):
    wid = lax.axis_index("s") * NC + lax.axis_index("c")
    base = wid * PPW

    pltpu.sync_copy(tk_hbm.at[pl.ds(base, PPW)], tk_v)
    lane = lax.broadcasted_iota(jnp.int32, (L,), 0)
    p_vec = base + lane
    b_vec = lax.shift_right_logical(p_vec, 3)
    flat_reg = b_vec * S + tk_v[...]
    half = lax.shift_right_logical(lane, 3)
    for t in range(PPW // 2):
        fvec = flat_reg.at[2 * t + half].get(mode="promise_in_bounds")
        idx_v[pl.ds(t * L, L)] = fvec

    def gissue(g):
        slot = g % NSLOT
        return pltpu.async_copy(
            mem_hbm.at[idx_v.at[pl.ds(g * RPB, 1)]], bufs[slot], gsems[slot])

    gcp = [None] * NSLOT
    ocp = [None] * NSLOT
    for g in range(NSLOT):
        gcp[g] = gissue(g)
    for g in range(NDMA):
        slot = g % NSLOT
        gcp[slot].wait()
        ocp[slot] = pltpu.async_copy(
            bufs[slot], out_hbm.at[pl.ds(base + g, 1)], osems[slot])
        if g + NSLOT < NDMA:
            ocp[slot].wait()
            gcp[slot] = gissue(g + NSLOT)
    for g in range(NDMA - NSLOT, NDMA):
        ocp[g % NSLOT].wait()


def kernel(memories_per_batch, topk_indices, gates):
    del gates
    mem6 = memories_per_batch.reshape(B, S, 2, 128, 8, 8)
    mem2 = mem6.transpose(0, 1, 5, 4, 2, 3).reshape(B * S, RPB, SUB)
    tk = topk_indices.reshape(P).astype(jnp.int32)

    mesh = plsc.VectorSubcoreMesh(
        core_axis_name="c", subcore_axis_name="s", num_cores=NC, num_subcores=NS
    )
    out = pl.kernel(
        _gather_body,
        out_type=jax.ShapeDtypeStruct((P, RPB, SUB), jnp.float32),
        mesh=mesh,
        scratch_types=[
            pltpu.VMEM((PPW,), jnp.int32),
            pltpu.VMEM((PPW * RPB,), jnp.int32),
            [pltpu.VMEM((1, RPB, SUB), jnp.float32) for _ in range(NSLOT)],
            [pltpu.SemaphoreType.DMA for _ in range(NSLOT)],
            [pltpu.SemaphoreType.DMA for _ in range(NSLOT)],
        ],
    )(mem2, tk)
    out6 = out.reshape(B, K, 8, 8, 2, 128)
    return out6.transpose(0, 1, 4, 5, 3, 2).reshape(B, K, H, D)
